# baseline (speedup 1.0000x reference)
import jax
import jax.numpy as jnp
from jax import lax
from jax.experimental import pallas as pl
from jax.experimental.pallas import tpu as pltpu
from jax.experimental.pallas import tpu_sc as plsc

POOLED = 7
SR = 2
SCALE = 112.0
H = 112
W = 112
C = 256
KPAD = 1024
NW = 32
RPW = KPAD // NW
NB = POOLED * POOLED


def _splat(v, lane):
    return v.at[jnp.full((16,), lane, jnp.int32)].get(mode="promise_in_bounds")


def _gather16(v, idx):
    return v.at[idx].get(mode="promise_in_bounds")


def _sc_body(f_hbm, rois_hbm, out_hbm, rois_v, idx_v, rows_v, outb_v, gsem, osem):
    wid = lax.axis_index("s") * 2 + lax.axis_index("c")
    cnt = jnp.where(wid < 8, RPW, RPW - 1)
    base_roi = wid * (RPW - 1) + jnp.minimum(wid, 8)
    abase = (base_roi // 8) * 8
    roff = base_roi - abase
    pltpu.sync_copy(rois_hbm.at[pl.ds(abase, RPW + 8)], rois_v)

    lanes = lax.iota(jnp.int32, 16)
    lanes_nb = lanes * NB
    iy_l = (lanes >> 3) & 1
    ix_l = (lanes >> 2) & 1
    cy_l = (lanes >> 1) & 1
    cx_l = lanes & 1
    p_f = (lanes >> 1).astype(jnp.float32)
    i_f = (lanes & 1).astype(jnp.float32) + 0.5

    def roi_geom(r):
        row = rois_v[r + roff]
        b_v = _splat(row, 0).astype(jnp.int32)
        x1 = _splat(row, 1) * SCALE
        y1 = _splat(row, 2) * SCALE
        x2 = _splat(row, 3) * SCALE
        y2 = _splat(row, 4) * SCALE
        roi_w = jnp.maximum(x2 - x1, 1.0)
        roi_h = jnp.maximum(y2 - y1, 1.0)
        bin_h = roi_h / POOLED
        bin_w = roi_w / POOLED
        base_row = b_v * (H * W)
        y = jnp.maximum(y1 + p_f * bin_h + i_f * bin_h / SR, 0.0)
        x = jnp.maximum(x1 + p_f * bin_w + i_f * bin_w / SR, 0.0)
        ylo = y.astype(jnp.int32)
        yhi = jnp.where(ylo >= H - 1, H - 1, ylo + 1)
        ylo = jnp.where(ylo >= H - 1, H - 1, ylo)
        ly = y - ylo.astype(jnp.float32)
        hy = 1.0 - ly
        xlo = x.astype(jnp.int32)
        xhi = jnp.where(xlo >= W - 1, W - 1, xlo + 1)
        xlo = jnp.where(xlo >= W - 1, W - 1, xlo)
        lx = x - xlo.astype(jnp.float32)
        hx = 1.0 - lx
        return base_row, ylo, yhi, ly, hy, xlo, xhi, lx, hx

    total = cnt * POOLED

    def fire(s, buf):
        r2 = s // POOLED
        p2 = lax.rem(s, POOLED)
        base_row, ylo, yhi, _, _, xlo, xhi, _, _ = roi_geom(r2)
        jy = iy_l + 2 * p2
        ytap = jnp.where(cy_l == 1, _gather16(yhi, jy), _gather16(ylo, jy))
        row_y = base_row + ytap * W
        for q in range(POOLED):
            jx = ix_l + 2 * q
            xtap = jnp.where(cx_l == 1, _gather16(xhi, jx), _gather16(xlo, jx))
            idx_v[buf, pl.ds(q * 16, 16)] = row_y + xtap
        pltpu.make_async_copy(f_hbm.at[idx_v.at[buf]], rows_v.at[buf], gsem).start()

    fire(0, 0)

    @pl.when(total > 1)
    def _():
        fire(1, 1)

    def s_body(s, _):
        buf = lax.rem(s, 3)
        r = s // POOLED
        p = lax.rem(s, POOLED)
        opar = lax.rem(r, 2)

        @pl.when(s + 2 < total)
        def _():
            fire(s + 2, lax.rem(s + 2, 3))

        @pl.when(jnp.logical_and(p == 0, r >= 2))
        def _():
            pltpu.make_async_copy(
                outb_v.at[opar], out_hbm.at[base_roi + r - 2], osem
            ).wait()

        pltpu.make_async_copy(
            f_hbm.at[idx_v.at[buf]], rows_v.at[buf], gsem
        ).wait()

        _, _, _, ly, hy, _, _, lx, hx = roi_geom(r)
        jy = iy_l + 2 * p
        wy = jnp.where(cy_l == 1, _gather16(ly, jy), _gather16(hy, jy))
        for q in range(POOLED):
            jx = ix_l + 2 * q
            wx = jnp.where(cx_l == 1, _gather16(lx, jx), _gather16(hx, jx))
            wq = wy * wx * (1.0 / (SR * SR))
            wsp = [_splat(wq, t) for t in range(16)]

            @plsc.parallel_loop(0, C // 32, unroll=8)
            def cc_body(cc, q=q, wsp=wsp, buf=buf, p=p):
                col = pl.ds(cc * 16, 16)
                aa0 = ab0 = aa1 = ab1 = None
                for t in range(16):
                    u = rows_v[buf, q * 16 + t, col]
                    a = lax.bitcast_convert_type(u << 16, jnp.float32)
                    b = lax.bitcast_convert_type(
                        u & jnp.int32(-65536), jnp.float32
                    )
                    if t == 0:
                        aa0, ab0 = wsp[0] * a, wsp[0] * b
                    elif t == 1:
                        aa1, ab1 = wsp[1] * a, wsp[1] * b
                    elif t % 2 == 0:
                        aa0 += wsp[t] * a
                        ab0 += wsp[t] * b
                    else:
                        aa1 += wsp[t] * a
                        ab1 += wsp[t] * b
                outb_v[opar, p * POOLED + q, pl.ds(cc * 32, 16)] = aa0 + aa1
                outb_v[opar, p * POOLED + q, pl.ds(cc * 32 + 16, 16)] = (
                    ab0 + ab1
                )

        @pl.when(p == POOLED - 1)
        def _():
            pltpu.make_async_copy(
                outb_v.at[opar], out_hbm.at[base_roi + r], osem
            ).start()

        return 0

    lax.fori_loop(0, total, s_body, 0)
    pltpu.make_async_copy(
        outb_v.at[0], out_hbm.at[base_roi + cnt - 2], osem
    ).wait()
    pltpu.make_async_copy(
        outb_v.at[1], out_hbm.at[base_roi + cnt - 1], osem
    ).wait()


def _pack_kernel(in_ref, out_ref):
    x = in_ref[0].reshape(C // 32, 2, 16, TW)
    a = x[:, 0].reshape(C // 2, TW).astype(jnp.bfloat16).astype(jnp.float32)
    b = x[:, 1].reshape(C // 2, TW).astype(jnp.bfloat16).astype(jnp.float32)
    au = lax.bitcast_convert_type(a, jnp.int32)
    bu = lax.bitcast_convert_type(b, jnp.int32)
    w = lax.shift_right_logical(au, 16) | (bu & jnp.int32(-65536))
    out_ref[...] = w.T


TW = 256


def _pack_table(input):
    n = input.shape[0]
    return pl.pallas_call(
        _pack_kernel,
        grid=(n, H * W // TW),
        in_specs=[
            pl.BlockSpec((1, C, TW), lambda b, i: (b, 0, i)),
        ],
        out_specs=pl.BlockSpec(
            (TW, C // 2), lambda b, i: (b * (H * W // TW) + i, 0)
        ),
        out_shape=jax.ShapeDtypeStruct((n * H * W, C // 2), jnp.int32),
    )(input.reshape(n, C, H * W))


def kernel(input, rois):
    n = input.shape[0]
    k = rois.shape[0]
    f = _pack_table(input)
    rois_p = jnp.pad(rois, ((0, 8), (0, 11)))
    mesh = plsc.VectorSubcoreMesh(core_axis_name="c", subcore_axis_name="s")
    run = pl.kernel(
        _sc_body,
        mesh=mesh,
        out_type=jax.ShapeDtypeStruct((k, NB, C), jnp.float32),
        scratch_types=[
            pltpu.VMEM((RPW + 8, 16), jnp.float32),
            pltpu.VMEM((3, POOLED * 16), jnp.int32),
            pltpu.VMEM((3, POOLED * 16, C // 2), jnp.int32),
            pltpu.VMEM((2, NB, C), jnp.float32),
            pltpu.SemaphoreType.DMA,
            pltpu.SemaphoreType.DMA,
        ],
    )
    out = run(f, rois_p)
    return jnp.transpose(
        out.reshape(k, POOLED, POOLED, C), (0, 3, 1, 2)
    )


# speedup vs baseline: 1.2832x; 1.2832x over previous
import jax
import jax.numpy as jnp
from jax import lax
from jax.experimental import pallas as pl
from jax.experimental.pallas import tpu as pltpu
from jax.experimental.pallas import tpu_sc as plsc

POOLED = 7
SR = 2
SCALE = 112.0
H = 112
W = 112
C = 256
KPAD = 1024
NW = 32
RPW = KPAD // NW
NB = POOLED * POOLED


def _splat(v, lane):
    return v.at[jnp.full((16,), lane, jnp.int32)].get(mode="promise_in_bounds")


def _gather16(v, idx):
    return v.at[idx].get(mode="promise_in_bounds")


def _sc_body(f_hbm, rois_hbm, out_hbm, rois_v, idx_v, rows_v, outb_v, gsem, osem):
    wid = lax.axis_index("s") * 2 + lax.axis_index("c")
    cnt = jnp.where(wid < 8, RPW, RPW - 1)
    base_roi = wid * (RPW - 1) + jnp.minimum(wid, 8)
    abase = (base_roi // 8) * 8
    roff = base_roi - abase
    pltpu.sync_copy(rois_hbm.at[pl.ds(abase, RPW + 8)], rois_v)

    lanes = lax.iota(jnp.int32, 16)
    lanes_nb = lanes * NB
    iy_l = (lanes >> 3) & 1
    ix_l = (lanes >> 2) & 1
    cy_l = (lanes >> 1) & 1
    cx_l = lanes & 1
    p_f = (lanes >> 1).astype(jnp.float32)
    i_f = (lanes & 1).astype(jnp.float32) + 0.5

    def roi_geom(r):
        row = rois_v[r + roff]
        b_v = _splat(row, 0).astype(jnp.int32)
        x1 = _splat(row, 1) * SCALE
        y1 = _splat(row, 2) * SCALE
        x2 = _splat(row, 3) * SCALE
        y2 = _splat(row, 4) * SCALE
        roi_w = jnp.maximum(x2 - x1, 1.0)
        roi_h = jnp.maximum(y2 - y1, 1.0)
        bin_h = roi_h / POOLED
        bin_w = roi_w / POOLED
        base_row = b_v * (H * W)
        y = jnp.maximum(y1 + p_f * bin_h + i_f * bin_h / SR, 0.0)
        x = jnp.maximum(x1 + p_f * bin_w + i_f * bin_w / SR, 0.0)
        ylo = y.astype(jnp.int32)
        yhi = jnp.where(ylo >= H - 1, H - 1, ylo + 1)
        ylo = jnp.where(ylo >= H - 1, H - 1, ylo)
        ly = y - ylo.astype(jnp.float32)
        hy = 1.0 - ly
        xlo = x.astype(jnp.int32)
        xhi = jnp.where(xlo >= W - 1, W - 1, xlo + 1)
        xlo = jnp.where(xlo >= W - 1, W - 1, xlo)
        lx = x - xlo.astype(jnp.float32)
        hx = 1.0 - lx
        return base_row, ylo, yhi, ly, hy, xlo, xhi, lx, hx

    total = cnt * POOLED

    def fire(s, buf):
        r2 = s // POOLED
        p2 = lax.rem(s, POOLED)
        base_row, ylo, yhi, _, _, xlo, xhi, _, _ = roi_geom(r2)
        jy = iy_l + 2 * p2
        ytap = jnp.where(cy_l == 1, _gather16(yhi, jy), _gather16(ylo, jy))
        row_y = base_row + ytap * W
        for q in range(POOLED):
            jx = ix_l + 2 * q
            xtap = jnp.where(cx_l == 1, _gather16(xhi, jx), _gather16(xlo, jx))
            idx_v[buf, pl.ds(q * 16, 16)] = row_y + xtap
        pltpu.make_async_copy(f_hbm.at[idx_v.at[buf]], rows_v.at[buf], gsem).start()

    fire(0, 0)

    @pl.when(total > 1)
    def _():
        fire(1, 1)

    def s_body(s, _):
        buf = lax.rem(s, 3)
        r = s // POOLED
        p = lax.rem(s, POOLED)
        opar = lax.rem(r, 2)

        @pl.when(s + 2 < total)
        def _():
            fire(s + 2, lax.rem(s + 2, 3))

        @pl.when(jnp.logical_and(p == 0, r >= 2))
        def _():
            pltpu.make_async_copy(
                outb_v.at[opar], out_hbm.at[base_roi + r - 2], osem
            ).wait()

        pltpu.make_async_copy(
            f_hbm.at[idx_v.at[buf]], rows_v.at[buf], gsem
        ).wait()

        _, _, _, ly, hy, _, _, lx, hx = roi_geom(r)
        jy = iy_l + 2 * p
        wy = jnp.where(cy_l == 1, _gather16(ly, jy), _gather16(hy, jy))
        for q in range(POOLED):
            jx = ix_l + 2 * q
            wx = jnp.where(cx_l == 1, _gather16(lx, jx), _gather16(hx, jx))
            wq = wy * wx * (1.0 / (SR * SR))
            wsp = [_splat(wq, t) for t in range(16)]

            @plsc.parallel_loop(0, C // 32, unroll=4)
            def cc_body(cc, q=q, wsp=wsp, buf=buf, p=p):
                col = pl.ds(cc * 16, 16)
                aa0 = ab0 = aa1 = ab1 = None
                for t in range(16):
                    u = rows_v[buf, q * 16 + t, col]
                    a = lax.bitcast_convert_type(u << 16, jnp.float32)
                    b = lax.bitcast_convert_type(u, jnp.float32)
                    if t == 0:
                        aa0, ab0 = wsp[0] * a, wsp[0] * b
                    elif t == 1:
                        aa1, ab1 = wsp[1] * a, wsp[1] * b
                    elif t % 2 == 0:
                        aa0 += wsp[t] * a
                        ab0 += wsp[t] * b
                    else:
                        aa1 += wsp[t] * a
                        ab1 += wsp[t] * b
                outb_v[opar, p * POOLED + q, pl.ds(cc * 32, 16)] = aa0 + aa1
                outb_v[opar, p * POOLED + q, pl.ds(cc * 32 + 16, 16)] = (
                    ab0 + ab1
                )

        @pl.when(p == POOLED - 1)
        def _():
            pltpu.make_async_copy(
                outb_v.at[opar], out_hbm.at[base_roi + r], osem
            ).start()

        return 0

    lax.fori_loop(0, total, s_body, 0)
    pltpu.make_async_copy(
        outb_v.at[0], out_hbm.at[base_roi + cnt - 2], osem
    ).wait()
    pltpu.make_async_copy(
        outb_v.at[1], out_hbm.at[base_roi + cnt - 1], osem
    ).wait()


def _pack_kernel(in_ref, out_ref):
    x = in_ref[0].reshape(C // 32, 2, 16, TW)
    a = x[:, 0].reshape(C // 2, TW).astype(jnp.bfloat16).astype(jnp.float32)
    b = x[:, 1].reshape(C // 2, TW).astype(jnp.bfloat16).astype(jnp.float32)
    au = lax.bitcast_convert_type(a, jnp.int32)
    bu = lax.bitcast_convert_type(b, jnp.int32)
    w = lax.shift_right_logical(au, 16) | (bu & jnp.int32(-65536))
    out_ref[...] = w.T


TW = 256


def _pack_table(input):
    n = input.shape[0]
    return pl.pallas_call(
        _pack_kernel,
        grid=(n, H * W // TW),
        in_specs=[
            pl.BlockSpec((1, C, TW), lambda b, i: (b, 0, i)),
        ],
        out_specs=pl.BlockSpec(
            (TW, C // 2), lambda b, i: (b * (H * W // TW) + i, 0)
        ),
        out_shape=jax.ShapeDtypeStruct((n * H * W, C // 2), jnp.int32),
    )(input.reshape(n, C, H * W))


def kernel(input, rois):
    n = input.shape[0]
    k = rois.shape[0]
    f = _pack_table(input)
    rois_p = jnp.pad(rois, ((0, 8), (0, 11)))
    mesh = plsc.VectorSubcoreMesh(core_axis_name="c", subcore_axis_name="s")
    run = pl.kernel(
        _sc_body,
        mesh=mesh,
        out_type=jax.ShapeDtypeStruct((k, NB, C), jnp.float32),
        scratch_types=[
            pltpu.VMEM((RPW + 8, 16), jnp.float32),
            pltpu.VMEM((3, POOLED * 16), jnp.int32),
            pltpu.VMEM((3, POOLED * 16, C // 2), jnp.int32),
            pltpu.VMEM((2, NB, C), jnp.float32),
            pltpu.SemaphoreType.DMA,
            pltpu.SemaphoreType.DMA,
        ],
    )
    out = run(f, rois_p)
    return jnp.transpose(
        out.reshape(k, POOLED, POOLED, C), (0, 3, 1, 2)
    )


# speedup vs baseline: 1.2835x; 1.0003x over previous
import jax
import jax.numpy as jnp
from jax import lax
from jax.experimental import pallas as pl
from jax.experimental.pallas import tpu as pltpu
from jax.experimental.pallas import tpu_sc as plsc

POOLED = 7
SR = 2
SCALE = 112.0
H = 112
W = 112
C = 256
NW = 32
RPW = 32
NB = POOLED * POOLED


def _splat(v, lane):
    return v.at[jnp.full((16,), lane, jnp.int32)].get(mode="promise_in_bounds")


def _gather16(v, idx):
    return v.at[idx].get(mode="promise_in_bounds")


def _sc_body(f_hbm, rois_hbm, out_hbm, rois_v, idx_v, rows_v, outb_v, gsem, osem):
    wid = lax.axis_index("s") * 2 + lax.axis_index("c")
    cnt = jnp.where(wid < 8, RPW, RPW - 1)
    base_roi = wid * (RPW - 1) + jnp.minimum(wid, 8)
    abase = (base_roi // 8) * 8
    roff = base_roi - abase
    pltpu.sync_copy(rois_hbm.at[pl.ds(abase, RPW + 8)], rois_v)

    lanes = lax.iota(jnp.int32, 16)
    iy_l = (lanes >> 3) & 1
    ix_l = (lanes >> 2) & 1
    cy_l = (lanes >> 1) & 1
    cx_l = lanes & 1
    p_f = (lanes >> 1).astype(jnp.float32)
    i_f = (lanes & 1).astype(jnp.float32) + 0.5

    def roi_geom(r):
        row = rois_v[r + roff]
        b_v = _splat(row, 0).astype(jnp.int32)
        x1 = _splat(row, 1) * SCALE
        y1 = _splat(row, 2) * SCALE
        x2 = _splat(row, 3) * SCALE
        y2 = _splat(row, 4) * SCALE
        roi_w = jnp.maximum(x2 - x1, 1.0)
        roi_h = jnp.maximum(y2 - y1, 1.0)
        bin_h = roi_h / POOLED
        bin_w = roi_w / POOLED
        base_row = b_v * (H * W)
        y = jnp.maximum(y1 + p_f * bin_h + i_f * bin_h / SR, 0.0)
        x = jnp.maximum(x1 + p_f * bin_w + i_f * bin_w / SR, 0.0)
        ylo = y.astype(jnp.int32)
        yhi = jnp.where(ylo >= H - 1, H - 1, ylo + 1)
        ylo = jnp.where(ylo >= H - 1, H - 1, ylo)
        ly = y - ylo.astype(jnp.float32)
        hy = 1.0 - ly
        xlo = x.astype(jnp.int32)
        xhi = jnp.where(xlo >= W - 1, W - 1, xlo + 1)
        xlo = jnp.where(xlo >= W - 1, W - 1, xlo)
        lx = x - xlo.astype(jnp.float32)
        hx = 1.0 - lx
        return base_row, ylo, yhi, ly, hy, xlo, xhi, lx, hx

    total = cnt * POOLED

    def fire(s, buf):
        r2 = s // POOLED
        p2 = lax.rem(s, POOLED)
        base_row, ylo, yhi, _, _, xlo, xhi, _, _ = roi_geom(r2)
        jy = iy_l + 2 * p2
        ytap = jnp.where(cy_l == 1, _gather16(yhi, jy), _gather16(ylo, jy))
        row_y = base_row + ytap * W
        for q in range(POOLED):
            jx = ix_l + 2 * q
            xtap = jnp.where(cx_l == 1, _gather16(xhi, jx), _gather16(xlo, jx))
            idx_v[buf, pl.ds(q * 16, 16)] = row_y + xtap
        pltpu.make_async_copy(f_hbm.at[idx_v.at[buf]], rows_v.at[buf], gsem).start()

    fire(0, 0)

    @pl.when(total > 1)
    def _():
        fire(1, 1)

    def s_body(s, _):
        buf = lax.rem(s, 3)
        r = s // POOLED
        p = lax.rem(s, POOLED)
        opar = lax.rem(r, 2)

        @pl.when(s + 2 < total)
        def _():
            fire(s + 2, lax.rem(s + 2, 3))

        @pl.when(jnp.logical_and(p == 0, r >= 2))
        def _():
            pltpu.make_async_copy(
                outb_v.at[opar], out_hbm.at[base_roi + r - 2], osem
            ).wait()

        pltpu.make_async_copy(
            f_hbm.at[idx_v.at[buf]], rows_v.at[buf], gsem
        ).wait()

        _, _, _, ly, hy, _, _, lx, hx = roi_geom(r)
        jy = iy_l + 2 * p
        wy = jnp.where(cy_l == 1, _gather16(ly, jy), _gather16(hy, jy))
        for q in range(POOLED):
            jx = ix_l + 2 * q
            wx = jnp.where(cx_l == 1, _gather16(lx, jx), _gather16(hx, jx))
            wq = wy * wx * (1.0 / (SR * SR))
            wsp = [_splat(wq, t) for t in range(16)]

            @plsc.parallel_loop(0, C // 32, unroll=4)
            def cc_body(cc, q=q, wsp=wsp, buf=buf, p=p):
                col = pl.ds(cc * 16, 16)
                aa0 = ab0 = aa1 = ab1 = None
                for t in range(16):
                    u = rows_v[buf, q * 16 + t, col]
                    a = lax.bitcast_convert_type(u << 16, jnp.float32)
                    b = lax.bitcast_convert_type(u, jnp.float32)
                    if t == 0:
                        aa0, ab0 = wsp[0] * a, wsp[0] * b
                    elif t == 1:
                        aa1, ab1 = wsp[1] * a, wsp[1] * b
                    elif t % 2 == 0:
                        aa0 += wsp[t] * a
                        ab0 += wsp[t] * b
                    else:
                        aa1 += wsp[t] * a
                        ab1 += wsp[t] * b
                outb_v[opar, p * POOLED + q, pl.ds(cc * 32, 16)] = aa0 + aa1
                outb_v[opar, p * POOLED + q, pl.ds(cc * 32 + 16, 16)] = (
                    ab0 + ab1
                )

        @pl.when(p == POOLED - 1)
        def _():
            pltpu.make_async_copy(
                outb_v.at[opar], out_hbm.at[base_roi + r], osem
            ).start()

        return 0

    lax.fori_loop(0, total, s_body, 0)
    pltpu.make_async_copy(
        outb_v.at[0], out_hbm.at[base_roi + cnt - 2], osem
    ).wait()
    pltpu.make_async_copy(
        outb_v.at[1], out_hbm.at[base_roi + cnt - 1], osem
    ).wait()


def _pack_kernel(in_ref, out_ref):
    x = in_ref[0].reshape(C // 32, 2, 16, TW)
    a = x[:, 0].reshape(C // 2, TW).astype(jnp.bfloat16).astype(jnp.float32)
    b = x[:, 1].reshape(C // 2, TW).astype(jnp.bfloat16).astype(jnp.float32)
    au = lax.bitcast_convert_type(a, jnp.int32)
    bu = lax.bitcast_convert_type(b, jnp.int32)
    w = lax.shift_right_logical(au, 16) | (bu & jnp.int32(-65536))
    out_ref[...] = w.T


TW = 256


def _pack_table(input):
    n = input.shape[0]
    return pl.pallas_call(
        _pack_kernel,
        grid=(n, H * W // TW),
        in_specs=[
            pl.BlockSpec((1, C, TW), lambda b, i: (b, 0, i)),
        ],
        out_specs=pl.BlockSpec(
            (TW, C // 2), lambda b, i: (b * (H * W // TW) + i, 0)
        ),
        out_shape=jax.ShapeDtypeStruct((n * H * W, C // 2), jnp.int32),
    )(input.reshape(n, C, H * W))


def kernel(input, rois):
    n = input.shape[0]
    k = rois.shape[0]
    f = _pack_table(input)
    rois_p = jnp.pad(rois, ((0, 8), (0, 11)))
    mesh = plsc.VectorSubcoreMesh(core_axis_name="c", subcore_axis_name="s")
    run = pl.kernel(
        _sc_body,
        mesh=mesh,
        out_type=jax.ShapeDtypeStruct((k, NB, C), jnp.float32),
        scratch_types=[
            pltpu.VMEM((RPW + 8, 16), jnp.float32),
            pltpu.VMEM((3, POOLED * 16), jnp.int32),
            pltpu.VMEM((3, POOLED * 16, C // 2), jnp.int32),
            pltpu.VMEM((2, NB, C), jnp.float32),
            pltpu.SemaphoreType.DMA,
            pltpu.SemaphoreType.DMA,
        ],
    )
    out = run(f, rois_p)
    return jnp.transpose(
        out.reshape(k, POOLED, POOLED, C), (0, 3, 1, 2)
    )
